# baseline (speedup 1.0000x reference)
import jax
import jax.numpy as jnp
from jax.experimental import pallas as pl

_N, _D = 1_000_000, 3
_BR = 8000


def _copy_body(x_ref, o_ref):
    pid = pl.program_id(0)
    vals = x_ref[...]

    @pl.when(pid == 0)
    def _():
        r = jax.lax.broadcasted_iota(jnp.int32, (_BR, _D), 0)
        c = jax.lax.broadcasted_iota(jnp.int32, (_BR, _D), 1).astype(jnp.float32)
        patched = jnp.where(r == 0, 10.0 + c, jnp.where(r == 2, 20.0 + c, vals))
        o_ref[...] = patched

    @pl.when(pid != 0)
    def _():
        o_ref[...] = vals


def kernel(x):
    return pl.pallas_call(
        _copy_body,
        grid=(_N // _BR,),
        in_specs=[pl.BlockSpec((_BR, _D), lambda i: (i, 0))],
        out_specs=pl.BlockSpec((_BR, _D), lambda i: (i, 0)),
        out_shape=jax.ShapeDtypeStruct((_N, _D), jnp.float32),
    )(x)
